# baseline (speedup 1.0000x reference)
import functools

import jax
import jax.numpy as jnp
from jax import lax
from jax.experimental import pallas as pl
from jax.experimental.pallas import tpu as pltpu
from jax.experimental.pallas import tpu_sc as plsc

N_IN = 100000
N_OUT = 100000
W = 128
LANES = 16
NW = 32
ROWS_PER_TILE = N_OUT // NW
CHUNK = 125
CHUNK_PAD = 128
NCHUNK = ROWS_PER_TILE // CHUNK
NROW = N_OUT // CHUNK


NBUF = 3


def _body(table, idx0, idx1, nmask, out,
          idx0_v, idx1_v, mask_v,
          buf_a0, buf_a1, buf_a2, buf_b0, buf_b1, buf_b2,
          insem0, insem1, insem2, outsem0, outsem1, outsem2):
    wid = lax.axis_index("s") * 2 + lax.axis_index("c")
    tile_row = wid * NCHUNK

    pltpu.sync_copy(idx0.at[wid], idx0_v)
    pltpu.sync_copy(idx1.at[wid], idx1_v)
    pltpu.sync_copy(nmask.at[wid], mask_v)

    buf_a = (buf_a0, buf_a1, buf_a2)
    buf_b = (buf_b0, buf_b1, buf_b2)
    insem = (insem0, insem1, insem2)
    outsem = (outsem0, outsem1, outsem2)

    H = CHUNK_PAD // 2

    def gather_parts(ci, s):
        return (
            (table.at[idx0_v.at[ci, pl.ds(0, H)]], buf_a[s].at[pl.ds(0, H)]),
            (table.at[idx0_v.at[ci, pl.ds(H, H)]], buf_a[s].at[pl.ds(H, H)]),
            (table.at[idx1_v.at[ci, pl.ds(0, H)]], buf_b[s].at[pl.ds(0, H)]),
            (table.at[idx1_v.at[ci, pl.ds(H, H)]], buf_b[s].at[pl.ds(H, H)]),
        )

    def start_gather(ci, s):
        for src, dst in gather_parts(ci, s):
            pltpu.async_copy(src, dst, insem[s])

    def wait_gather(ci, s):
        for src, dst in gather_parts(ci, s):
            pltpu.make_async_copy(src, dst, insem[s]).wait()

    def out_slice(ci):
        return out.at[tile_row + ci]

    def start_out(ci, s):
        pltpu.async_copy(buf_a[s].at[pl.ds(0, CHUNK)], out_slice(ci), outsem[s])

    def wait_out(ci, s):
        pltpu.make_async_copy(buf_a[s].at[pl.ds(0, CHUNK)], out_slice(ci),
                              outsem[s]).wait()

    def compute(ci, s):
        a_ref, b_ref = buf_a[s], buf_b[s]

        def g_body(g, carry):
            base = g * LANES
            m16 = mask_v[ci, pl.ds(base, LANES)]
            for l in range(LANES):
                m = jnp.full((LANES,), m16[l], jnp.int32)
                for w in range(W // LANES):
                    a = a_ref[base + l, pl.ds(w * LANES, LANES)]
                    b = b_ref[base + l, pl.ds(w * LANES, LANES)]
                    a_ref[base + l, pl.ds(w * LANES, LANES)] = (
                        ~((a & b) ^ (m & (a ^ b))))
            return carry

        lax.fori_loop(0, CHUNK_PAD // LANES, g_body, 0)

    start_gather(0, 0)
    start_gather(1, 1)

    def tri_body(p, carry):
        for b in range(NBUF):
            ci = NBUF * p + b

            @pl.when(ci < NCHUNK)
            def _(ci=ci, b=b):
                s2 = (b + 2) % NBUF

                @pl.when(ci + 2 < NCHUNK)
                def _():
                    @pl.when(ci >= 1)
                    def _():
                        wait_out(ci - 1, s2)

                    start_gather(ci + 2, s2)

                wait_gather(ci, b)
                compute(ci, b)
                start_out(ci, b)
        return carry

    lax.fori_loop(0, (NCHUNK + NBUF - 1) // NBUF, tri_body, 0)
    for ci in range(NCHUNK - 3, NCHUNK):
        wait_out(ci, ci % NBUF)


@jax.jit
def _nand_layer(table, idx0, idx1, nmask):
    mesh = plsc.VectorSubcoreMesh(core_axis_name="c", subcore_axis_name="s")
    f = functools.partial(
        pl.kernel,
        out_type=jax.ShapeDtypeStruct((NROW, CHUNK, W), jnp.int32),
        mesh=mesh,
        scratch_types=[
            pltpu.VMEM((NCHUNK, CHUNK_PAD), jnp.int32),
            pltpu.VMEM((NCHUNK, CHUNK_PAD), jnp.int32),
            pltpu.VMEM((NCHUNK, CHUNK_PAD), jnp.int32),
            pltpu.VMEM((CHUNK_PAD, W), jnp.int32),
            pltpu.VMEM((CHUNK_PAD, W), jnp.int32),
            pltpu.VMEM((CHUNK_PAD, W), jnp.int32),
            pltpu.VMEM((CHUNK_PAD, W), jnp.int32),
            pltpu.VMEM((CHUNK_PAD, W), jnp.int32),
            pltpu.VMEM((CHUNK_PAD, W), jnp.int32),
            pltpu.SemaphoreType.DMA,
            pltpu.SemaphoreType.DMA,
            pltpu.SemaphoreType.DMA,
            pltpu.SemaphoreType.DMA,
            pltpu.SemaphoreType.DMA,
            pltpu.SemaphoreType.DMA,
        ],
    )(_body)
    return f(table, idx0, idx1, nmask)


def _chunk_layout(x):
    x = x.reshape(NW, NCHUNK, CHUNK)
    return jnp.pad(x, ((0, 0), (0, 0), (0, CHUNK_PAD - CHUNK)))


def kernel(input_bitarrays, output_node_input_indices, nor_mask):
    flat = output_node_input_indices.astype(jnp.int32).reshape(-1)
    idx0 = _chunk_layout(flat[0::2])
    idx1 = _chunk_layout(flat[1::2])
    nmask = _chunk_layout(jnp.where(nor_mask, jnp.int32(-1), jnp.int32(0)))
    out = _nand_layer(input_bitarrays, idx0, idx1, nmask)
    return out.reshape(N_OUT, W)


# speedup vs baseline: 1.6976x; 1.6976x over previous
import functools

import jax
import jax.numpy as jnp
from jax import lax
from jax.experimental import pallas as pl
from jax.experimental.pallas import tpu as pltpu
from jax.experimental.pallas import tpu_sc as plsc

N_IN = 100000
N_OUT = 100000
W = 128
LANES = 16
NW = 32
G = 64
GI = 2 * G
NCH_FULL = N_OUT // G
TAIL = N_OUT - NCH_FULL * G
NCH_HI = 49
CUT = NCH_FULL - 48 * NW
NBUF = 3


def _body(table, flat_idx, nmask, out,
          idx_v, mask_v,
          buf0, buf1, buf2, ob0, ob1, ob2,
          insem0, insem1, insem2, outsem0, outsem1, outsem2):
    t = lax.axis_index("s") * 2 + lax.axis_index("c")
    nch = jnp.where(t < CUT, NCH_HI, NCH_HI - 1)
    c0 = NCH_HI * t - jnp.maximum(t - CUT, 0)

    i_base = pl.multiple_of(c0 * GI, GI)
    m_base = pl.multiple_of(c0 * G, G)

    @pl.when(t < CUT)
    def _():
        pltpu.sync_copy(flat_idx.at[pl.ds(i_base, NCH_HI * GI)],
                        idx_v.at[pl.ds(0, NCH_HI * GI)])
        pltpu.sync_copy(nmask.at[pl.ds(m_base, NCH_HI * G)],
                        mask_v.at[pl.ds(0, NCH_HI * G)])

    @pl.when(jnp.logical_and(t >= CUT, t < NW - 1))
    def _():
        pltpu.sync_copy(flat_idx.at[pl.ds(i_base, 48 * GI)],
                        idx_v.at[pl.ds(0, 48 * GI)])
        pltpu.sync_copy(nmask.at[pl.ds(m_base, 48 * G)],
                        mask_v.at[pl.ds(0, 48 * G)])

    @pl.when(t == NW - 1)
    def _():
        pltpu.sync_copy(flat_idx.at[pl.ds(i_base, 48 * GI + 2 * TAIL)],
                        idx_v.at[pl.ds(0, 48 * GI + 2 * TAIL)])
        pltpu.sync_copy(nmask.at[pl.ds(m_base, 48 * G + TAIL)],
                        mask_v.at[pl.ds(0, 48 * G + TAIL)])

    buf = (buf0, buf1, buf2)
    ob = (ob0, ob1, ob2)
    insem = (insem0, insem1, insem2)
    outsem = (outsem0, outsem1, outsem2)

    def idx_slice(ci):
        return idx_v.at[pl.ds(pl.multiple_of(ci * GI, GI), GI)]

    def start_gather(ci, s):
        pltpu.async_copy(table.at[idx_slice(ci)], buf[s], insem[s])

    def wait_gather(ci, s):
        pltpu.make_async_copy(table.at[idx_slice(ci)], buf[s], insem[s]).wait()

    def out_slice(ci):
        return out.at[pl.ds(pl.multiple_of((c0 + ci) * G, G), G)]

    def start_out(ci, s):
        pltpu.async_copy(ob[s], out_slice(ci), outsem[s])

    def wait_out(ci, s):
        pltpu.make_async_copy(ob[s], out_slice(ci), outsem[s]).wait()

    def compute(ci, s, ngroups=G // LANES):
        a_ref, o_ref = buf[s], ob[s]

        def g_body(g, carry):
            base = g * LANES
            m16 = mask_v[pl.ds(pl.multiple_of(ci * G + base, LANES), LANES)]
            for l in range(LANES):
                r = base + l
                m = jnp.full((LANES,), m16[l], jnp.int32)
                for w in range(W // LANES):
                    a = a_ref[2 * r, pl.ds(w * LANES, LANES)]
                    b = a_ref[2 * r + 1, pl.ds(w * LANES, LANES)]
                    o_ref[r, pl.ds(w * LANES, LANES)] = (
                        ~((a & b) ^ (m & (a ^ b))))
            return carry

        lax.fori_loop(0, ngroups, g_body, 0)

    start_gather(0, 0)
    start_gather(1, 1)

    def tri_body(p, carry):
        for b in range(NBUF):
            ci = NBUF * p + b

            @pl.when(ci < nch)
            def _(ci=ci, b=b):
                s2 = (b + 2) % NBUF

                @pl.when(ci + 2 < nch)
                def _():
                    @pl.when(ci >= 1)
                    def _():
                        wait_out(ci - 1, s2)

                    start_gather(ci + 2, s2)

                wait_gather(ci, b)
                compute(ci, b)
                start_out(ci, b)
        return carry

    lax.fori_loop(0, (NCH_HI + NBUF - 1) // NBUF, tri_body, 0)
    for b in range(NBUF):
        last = nch - 1 - ((nch - 1 - b) % NBUF)
        wait_out(last, b)

    @pl.when(t == NW - 1)
    def _():
        pltpu.async_copy(
            table.at[idx_v.at[pl.ds(48 * GI, 2 * TAIL)]],
            buf0.at[pl.ds(0, 2 * TAIL)], insem0)
        pltpu.make_async_copy(
            table.at[idx_v.at[pl.ds(48 * GI, 2 * TAIL)]],
            buf0.at[pl.ds(0, 2 * TAIL)], insem0).wait()
        compute(48, 0, ngroups=TAIL // LANES)
        pltpu.async_copy(ob0.at[pl.ds(0, TAIL)],
                         out.at[pl.ds(NCH_FULL * G, TAIL)], outsem0)
        pltpu.make_async_copy(ob0.at[pl.ds(0, TAIL)],
                              out.at[pl.ds(NCH_FULL * G, TAIL)],
                              outsem0).wait()


@jax.jit
def _nand_layer(table, flat_idx, nmask):
    mesh = plsc.VectorSubcoreMesh(core_axis_name="c", subcore_axis_name="s")
    f = functools.partial(
        pl.kernel,
        out_type=jax.ShapeDtypeStruct((N_OUT, W), jnp.int32),
        mesh=mesh,
        scratch_types=[
            pltpu.VMEM((NCH_HI * GI,), jnp.int32),
            pltpu.VMEM((NCH_HI * G,), jnp.int32),
            pltpu.VMEM((GI, W), jnp.int32),
            pltpu.VMEM((GI, W), jnp.int32),
            pltpu.VMEM((GI, W), jnp.int32),
            pltpu.VMEM((G, W), jnp.int32),
            pltpu.VMEM((G, W), jnp.int32),
            pltpu.VMEM((G, W), jnp.int32),
            pltpu.SemaphoreType.DMA,
            pltpu.SemaphoreType.DMA,
            pltpu.SemaphoreType.DMA,
            pltpu.SemaphoreType.DMA,
            pltpu.SemaphoreType.DMA,
            pltpu.SemaphoreType.DMA,
        ],
    )(_body)
    return f(table, flat_idx, nmask)


def kernel(input_bitarrays, output_node_input_indices, nor_mask):
    flat = output_node_input_indices.astype(jnp.int32).reshape(-1)
    nmask = jnp.where(nor_mask, jnp.int32(-1), jnp.int32(0))
    return _nand_layer(input_bitarrays, flat, nmask)


# speedup vs baseline: 5.6198x; 3.3104x over previous
import functools

import jax
import jax.numpy as jnp
from jax import lax
from jax.experimental import pallas as pl
from jax.experimental.pallas import tpu as pltpu
from jax.experimental.pallas import tpu_sc as plsc

N_IN = 100000
N_OUT = 100000
W = 128
LANES = 16
NW = 32
G = 64
GI = 2 * G
NCH_FULL = N_OUT // G
TAIL = N_OUT - NCH_FULL * G
NCH_HI = 49
CUT = NCH_FULL - 48 * NW
HALF = NCH_HI * G
NBUF = 3


def _body(table, idx0, idx1, nmask, out,
          idx_v, mask_v,
          buf0, buf1, buf2, ob0, ob1, ob2,
          insem0, insem1, insem2, outsem0, outsem1, outsem2):
    t = lax.axis_index("s") * 2 + lax.axis_index("c")
    nch = jnp.where(t < CUT, NCH_HI, NCH_HI - 1)
    c0 = NCH_HI * t - jnp.maximum(t - CUT, 0)

    m_base = pl.multiple_of(c0 * G, G)

    def stage(n):
        pltpu.sync_copy(idx0.at[pl.ds(m_base, n)], idx_v.at[pl.ds(0, n)])
        pltpu.sync_copy(idx1.at[pl.ds(m_base, n)], idx_v.at[pl.ds(HALF, n)])
        pltpu.sync_copy(nmask.at[pl.ds(m_base, n)], mask_v.at[pl.ds(0, n)])

    @pl.when(t < CUT)
    def _():
        stage(NCH_HI * G)

    @pl.when(jnp.logical_and(t >= CUT, t < NW - 1))
    def _():
        stage(48 * G)

    @pl.when(t == NW - 1)
    def _():
        stage(48 * G + TAIL)

    buf = (buf0, buf1, buf2)
    ob = (ob0, ob1, ob2)
    insem = (insem0, insem1, insem2)
    outsem = (outsem0, outsem1, outsem2)

    def gather_parts(ci, s):
        a_off = pl.multiple_of(ci * G, G)
        return (
            (table.at[idx_v.at[pl.ds(a_off, G)]], buf[s].at[pl.ds(0, G)]),
            (table.at[idx_v.at[pl.ds(HALF + a_off, G)]],
             buf[s].at[pl.ds(G, G)]),
        )

    def start_gather(ci, s):
        for src_, dst in gather_parts(ci, s):
            pltpu.async_copy(src_, dst, insem[s])

    def wait_gather(ci, s):
        for src_, dst in gather_parts(ci, s):
            pltpu.make_async_copy(src_, dst, insem[s]).wait()

    def out_slice(ci):
        return out.at[pl.ds(pl.multiple_of((c0 + ci) * G, G), G)]

    def start_out(ci, s):
        pltpu.async_copy(ob[s], out_slice(ci), outsem[s])

    def wait_out(ci, s):
        pltpu.make_async_copy(ob[s], out_slice(ci), outsem[s]).wait()

    def compute(ci, s, ngroups=G // LANES):
        a_ref, o_ref = buf[s], ob[s]

        def g_body(g, carry):
            base = g * LANES
            m16 = mask_v[pl.ds(pl.multiple_of(ci * G + base, LANES), LANES)]
            for l in range(LANES):
                r = base + l
                m = jnp.full((LANES,), m16[l], jnp.int32)
                for w in range(W // LANES):
                    a = a_ref[r, pl.ds(w * LANES, LANES)]
                    b = a_ref[G + r, pl.ds(w * LANES, LANES)]
                    o_ref[r, pl.ds(w * LANES, LANES)] = (
                        ~((a & b) ^ (m & (a ^ b))))
            return carry

        lax.fori_loop(0, ngroups, g_body, 0)

    start_gather(0, 0)
    start_gather(1, 1)

    def tri_body(p, carry):
        for b in range(NBUF):
            ci = NBUF * p + b

            @pl.when(ci < nch)
            def _(ci=ci, b=b):
                s2 = (b + 2) % NBUF

                @pl.when(ci + 2 < nch)
                def _():
                    @pl.when(ci >= 1)
                    def _():
                        wait_out(ci - 1, s2)

                    start_gather(ci + 2, s2)

                wait_gather(ci, b)
                compute(ci, b)
                start_out(ci, b)
        return carry

    lax.fori_loop(0, (NCH_HI + NBUF - 1) // NBUF, tri_body, 0)
    for b in range(NBUF):
        last = nch - 1 - ((nch - 1 - b) % NBUF)
        wait_out(last, b)

    @pl.when(t == NW - 1)
    def _():
        pltpu.async_copy(
            table.at[idx_v.at[pl.ds(48 * G, TAIL)]],
            buf0.at[pl.ds(0, TAIL)], insem0)
        pltpu.async_copy(
            table.at[idx_v.at[pl.ds(HALF + 48 * G, TAIL)]],
            buf0.at[pl.ds(G, TAIL)], insem0)
        pltpu.make_async_copy(
            table.at[idx_v.at[pl.ds(48 * G, TAIL)]],
            buf0.at[pl.ds(0, TAIL)], insem0).wait()
        pltpu.make_async_copy(
            table.at[idx_v.at[pl.ds(HALF + 48 * G, TAIL)]],
            buf0.at[pl.ds(G, TAIL)], insem0).wait()
        compute(48, 0, ngroups=TAIL // LANES)
        pltpu.async_copy(ob0.at[pl.ds(0, TAIL)],
                         out.at[pl.ds(NCH_FULL * G, TAIL)], outsem0)
        pltpu.make_async_copy(ob0.at[pl.ds(0, TAIL)],
                              out.at[pl.ds(NCH_FULL * G, TAIL)],
                              outsem0).wait()


@jax.jit
def _nand_layer(table, idx0, idx1, nmask):
    mesh = plsc.VectorSubcoreMesh(core_axis_name="c", subcore_axis_name="s")
    f = functools.partial(
        pl.kernel,
        out_type=jax.ShapeDtypeStruct((N_OUT, W), jnp.int32),
        mesh=mesh,
        scratch_types=[
            pltpu.VMEM((NCH_HI * GI,), jnp.int32),
            pltpu.VMEM((NCH_HI * G,), jnp.int32),
            pltpu.VMEM((GI, W), jnp.int32),
            pltpu.VMEM((GI, W), jnp.int32),
            pltpu.VMEM((GI, W), jnp.int32),
            pltpu.VMEM((G, W), jnp.int32),
            pltpu.VMEM((G, W), jnp.int32),
            pltpu.VMEM((G, W), jnp.int32),
            pltpu.SemaphoreType.DMA,
            pltpu.SemaphoreType.DMA,
            pltpu.SemaphoreType.DMA,
            pltpu.SemaphoreType.DMA,
            pltpu.SemaphoreType.DMA,
            pltpu.SemaphoreType.DMA,
        ],
    )(_body)
    return f(table, idx0, idx1, nmask)


def kernel(input_bitarrays, output_node_input_indices, nor_mask):
    idx = output_node_input_indices.astype(jnp.int32)
    nmask = jnp.where(nor_mask, jnp.int32(-1), jnp.int32(0))
    return _nand_layer(input_bitarrays, idx[:, 0], idx[:, 1], nmask)


# speedup vs baseline: 5.6485x; 1.0051x over previous
import functools

import jax
import jax.numpy as jnp
from jax import lax
from jax.experimental import pallas as pl
from jax.experimental.pallas import tpu as pltpu
from jax.experimental.pallas import tpu_sc as plsc

N_IN = 100000
N_OUT = 100000
W = 128
LANES = 16
NW = 32
G = 64
GI = 2 * G
NCH_FULL = N_OUT // G
TAIL = N_OUT - NCH_FULL * G
NCH_HI = 49
CUT = NCH_FULL - 48 * NW
HALF = NCH_HI * G
NBUF = 4


def _body(table, idx0, idx1, nmask, out,
          idx_v, mask_v,
          buf0, buf1, buf2, buf3, ob0, ob1, ob2, ob3,
          insem0, insem1, insem2, insem3,
          outsem0, outsem1, outsem2, outsem3):
    t = lax.axis_index("s") * 2 + lax.axis_index("c")
    nch = jnp.where(t < CUT, NCH_HI, NCH_HI - 1)
    c0 = NCH_HI * t - jnp.maximum(t - CUT, 0)

    m_base = pl.multiple_of(c0 * G, G)

    def stage(n):
        pltpu.sync_copy(idx0.at[pl.ds(m_base, n)], idx_v.at[pl.ds(0, n)])
        pltpu.sync_copy(idx1.at[pl.ds(m_base, n)], idx_v.at[pl.ds(HALF, n)])
        pltpu.sync_copy(nmask.at[pl.ds(m_base, n)], mask_v.at[pl.ds(0, n)])

    @pl.when(t < CUT)
    def _():
        stage(NCH_HI * G)

    @pl.when(jnp.logical_and(t >= CUT, t < NW - 1))
    def _():
        stage(48 * G)

    @pl.when(t == NW - 1)
    def _():
        stage(48 * G + TAIL)

    buf = (buf0, buf1, buf2, buf3)
    ob = (ob0, ob1, ob2, ob3)
    insem = (insem0, insem1, insem2, insem3)
    outsem = (outsem0, outsem1, outsem2, outsem3)

    def gather_parts(ci, s):
        a_off = pl.multiple_of(ci * G, G)
        return (
            (table.at[idx_v.at[pl.ds(a_off, G)]], buf[s].at[pl.ds(0, G)]),
            (table.at[idx_v.at[pl.ds(HALF + a_off, G)]],
             buf[s].at[pl.ds(G, G)]),
        )

    def start_gather(ci, s):
        for src_, dst in gather_parts(ci, s):
            pltpu.async_copy(src_, dst, insem[s])

    def wait_gather(ci, s):
        for src_, dst in gather_parts(ci, s):
            pltpu.make_async_copy(src_, dst, insem[s]).wait()

    def out_slice(ci):
        return out.at[pl.ds(pl.multiple_of((c0 + ci) * G, G), G)]

    def start_out(ci, s):
        pltpu.async_copy(ob[s], out_slice(ci), outsem[s])

    def wait_out(ci, s):
        pltpu.make_async_copy(ob[s], out_slice(ci), outsem[s]).wait()

    def compute(ci, s, ngroups=G // LANES):
        a_ref, o_ref = buf[s], ob[s]

        def g_body(g, carry):
            base = g * LANES
            m16 = mask_v[pl.ds(pl.multiple_of(ci * G + base, LANES), LANES)]
            for l in range(LANES):
                r = base + l
                m = jnp.full((LANES,), m16[l], jnp.int32)
                for w in range(W // LANES):
                    a = a_ref[r, pl.ds(w * LANES, LANES)]
                    b = a_ref[G + r, pl.ds(w * LANES, LANES)]
                    o_ref[r, pl.ds(w * LANES, LANES)] = (
                        ~((a & b) ^ (m & (a ^ b))))
            return carry

        lax.fori_loop(0, ngroups, g_body, 0)

    start_gather(0, 0)
    start_gather(1, 1)
    start_gather(2, 2)

    def ring_body(p, carry):
        for b in range(NBUF):
            ci = NBUF * p + b

            @pl.when(ci < nch)
            def _(ci=ci, b=b):
                s3 = (b + 3) % NBUF

                @pl.when(ci + 3 < nch)
                def _():
                    @pl.when(ci >= 1)
                    def _():
                        wait_out(ci - 1, s3)

                    start_gather(ci + 3, s3)

                wait_gather(ci, b)
                compute(ci, b)
                start_out(ci, b)
        return carry

    lax.fori_loop(0, (NCH_HI + NBUF - 1) // NBUF, ring_body, 0)
    for b in range(NBUF):
        last = nch - 1 - ((nch - 1 - b) % NBUF)
        wait_out(last, b)

    @pl.when(t == NW - 1)
    def _():
        pltpu.async_copy(
            table.at[idx_v.at[pl.ds(48 * G, TAIL)]],
            buf0.at[pl.ds(0, TAIL)], insem0)
        pltpu.async_copy(
            table.at[idx_v.at[pl.ds(HALF + 48 * G, TAIL)]],
            buf0.at[pl.ds(G, TAIL)], insem0)
        pltpu.make_async_copy(
            table.at[idx_v.at[pl.ds(48 * G, TAIL)]],
            buf0.at[pl.ds(0, TAIL)], insem0).wait()
        pltpu.make_async_copy(
            table.at[idx_v.at[pl.ds(HALF + 48 * G, TAIL)]],
            buf0.at[pl.ds(G, TAIL)], insem0).wait()
        compute(48, 0, ngroups=TAIL // LANES)
        pltpu.async_copy(ob0.at[pl.ds(0, TAIL)],
                         out.at[pl.ds(NCH_FULL * G, TAIL)], outsem0)
        pltpu.make_async_copy(ob0.at[pl.ds(0, TAIL)],
                              out.at[pl.ds(NCH_FULL * G, TAIL)],
                              outsem0).wait()


@jax.jit
def _nand_layer(table, idx0, idx1, nmask):
    mesh = plsc.VectorSubcoreMesh(core_axis_name="c", subcore_axis_name="s")
    f = functools.partial(
        pl.kernel,
        out_type=jax.ShapeDtypeStruct((N_OUT, W), jnp.int32),
        mesh=mesh,
        scratch_types=[
            pltpu.VMEM((NCH_HI * GI,), jnp.int32),
            pltpu.VMEM((NCH_HI * G,), jnp.int32),
            pltpu.VMEM((GI, W), jnp.int32),
            pltpu.VMEM((GI, W), jnp.int32),
            pltpu.VMEM((GI, W), jnp.int32),
            pltpu.VMEM((GI, W), jnp.int32),
            pltpu.VMEM((G, W), jnp.int32),
            pltpu.VMEM((G, W), jnp.int32),
            pltpu.VMEM((G, W), jnp.int32),
            pltpu.VMEM((G, W), jnp.int32),
            pltpu.SemaphoreType.DMA,
            pltpu.SemaphoreType.DMA,
            pltpu.SemaphoreType.DMA,
            pltpu.SemaphoreType.DMA,
            pltpu.SemaphoreType.DMA,
            pltpu.SemaphoreType.DMA,
            pltpu.SemaphoreType.DMA,
            pltpu.SemaphoreType.DMA,
        ],
    )(_body)
    return f(table, idx0, idx1, nmask)


def kernel(input_bitarrays, output_node_input_indices, nor_mask):
    idx = output_node_input_indices.astype(jnp.int32)
    nmask = jnp.where(nor_mask, jnp.int32(-1), jnp.int32(0))
    return _nand_layer(input_bitarrays, idx[:, 0], idx[:, 1], nmask)
